# baseline (speedup 1.0000x reference)
import math

import jax
import jax.numpy as jnp
from jax import lax
from jax.experimental import pallas as pl
from jax.experimental.pallas import tpu as pltpu

C = 256
HF = 50
WF = 50
P = 7
KW = 12
RB = 8
KWW = 32
WP = 64
NPROP = 1000
NPAD = 1024
NC = 91
FC = 1024
KDIM = C * P * P
BM = 256
KT = 7
BK = KDIM // KT
NEG = -1e30
BBOX_CLIP = math.log(1000.0 / 16)


def _pool_kernel(b_ref, f_ref, o_ref, out1_ref, pyr_ref):
    @pl.when(pl.program_id(0) == 0)
    def _build():
        def cp(h, _):
            pyr_ref[pl.ds(h, 1), :, :] = f_ref[pl.ds(h, 1), :, :]
            return 0
        lax.fori_loop(0, HF, cp, 0)
        for k in (1, 2, 3):
            d = 1 << (k - 1)

            def bd(h, _, base=k * HF, pbase=(k - 1) * HF, d=d):
                a = pyr_ref[pl.ds(pbase + h, 1), :, :]
                b = pyr_ref[pl.ds(pbase + jnp.minimum(h + d, HF - 1), 1), :, :]
                pyr_ref[pl.ds(base + h, 1), :, :] = jnp.maximum(a, b)
                return 0

            lax.fori_loop(0, HF, bd, 0)

    def roi_body(i, _):
        for ph in range(P):
            ra = b_ref[i, 0, ph]
            rb = b_ref[i, 0, P + ph]
            ln = b_ref[i, 0, 2 * P + ph]
            a = pyr_ref[pl.ds(ra, 1), :, :]
            b = pyr_ref[pl.ds(rb, 1), :, :]
            v = jnp.maximum(a, b)[0]
            v = jnp.where(ln > 0, v, NEG)
            out1_ref[ph, 0:WF, :] = v
        for pw in range(P):
            sw = b_ref[i, 0, 3 * P + pw] * 16
            lo = b_ref[i, 0, 4 * P + pw]
            hi = b_ref[i, 0, 5 * P + pw]
            win = out1_ref[:, pl.ds(sw, KWW), :]
            r = lax.broadcasted_iota(jnp.int32, (1, KWW, 1), 1)
            v = jnp.where((r >= lo) & (r < hi), win, NEG)
            cell = jnp.max(v, axis=1)
            cell = jnp.where(cell < -1e29, 0.0, cell)
            o_ref[pl.ds(i, 1), pw * P:(pw + 1) * P, :] = cell[None]
        return 0

    lax.fori_loop(0, RB, roi_body, 0)


def _fc_kernel(x_ref, w6_ref, b6_ref, w7_ref, b7_ref, wh_ref, bh_ref, pr_ref,
               sc_ref, x1_ref, y1_ref, x2_ref, y2_ref, acc_ref):
    k = pl.program_id(1)
    dn = (((1,), (1,)), ((), ()))

    @pl.when(k == 0)
    def _():
        acc_ref[:] = jnp.zeros_like(acc_ref)

    acc_ref[:] += lax.dot_general(x_ref[:], w6_ref[:], dn,
                                  preferred_element_type=jnp.float32)

    @pl.when(k == KT - 1)
    def _():
        h6 = jnp.maximum(acc_ref[:] + b6_ref[:], 0.0).astype(jnp.bfloat16)
        h7 = lax.dot_general(h6, w7_ref[:], dn,
                             preferred_element_type=jnp.float32) + b7_ref[:]
        h7 = jnp.maximum(h7, 0.0).astype(jnp.bfloat16)
        head = lax.dot_general(h7, wh_ref[:], dn,
                               preferred_element_type=jnp.float32) + bh_ref[:]
        cls = head[:, 0:NC]
        mx = jnp.max(cls, axis=1, keepdims=True)
        e = jnp.exp(cls - mx)
        sc_ref[:] = e / jnp.sum(e, axis=1, keepdims=True)
        dx = head[:, NC:2 * NC]
        dy = head[:, 2 * NC:3 * NC]
        dw = jnp.minimum(head[:, 3 * NC:4 * NC], BBOX_CLIP)
        dh = jnp.minimum(head[:, 4 * NC:5 * NC], BBOX_CLIP)
        p = pr_ref[:]
        w = p[:, 2:3] - p[:, 0:1]
        h = p[:, 3:4] - p[:, 1:2]
        cx = p[:, 0:1] + w * 0.5
        cy = p[:, 1:2] + h * 0.5
        px = dx * w + cx
        py = dy * h + cy
        pw = jnp.exp(dw) * w
        ph = jnp.exp(dh) * h
        x1_ref[:] = px - 0.5 * pw
        y1_ref[:] = py - 0.5 * ph
        x2_ref[:] = px + 0.5 * pw
        y2_ref[:] = py + 0.5 * ph


def kernel(feat, proposals, image_shape, W6, b6, W7, b7, Wc, bc, Wb, bb):
    fs = jnp.array(feat.shape[-2:], dtype=jnp.float32)
    ms = image_shape.astype(jnp.float32)
    scale = (2.0 ** jnp.round(jnp.log2(fs / ms)))[0]
    r = jnp.floor(proposals * scale + 0.5)
    x1 = r[:, 0]
    y1 = r[:, 1]
    roi_w = jnp.maximum(r[:, 2] - x1 + 1.0, 1.0)
    roi_h = jnp.maximum(r[:, 3] - y1 + 1.0, 1.0)
    bw = roi_w / P
    bh = roi_h / P
    j = jnp.arange(P, dtype=jnp.float32)
    ws = jnp.clip(jnp.floor(j * bw[:, None]) + x1[:, None], 0.0, WF).astype(jnp.int32)
    we = jnp.clip(jnp.ceil((j + 1.0) * bw[:, None]) + x1[:, None], 0.0, WF).astype(jnp.int32)
    hs = jnp.clip(jnp.floor(j * bh[:, None]) + y1[:, None], 0.0, HF).astype(jnp.int32)
    he = jnp.clip(jnp.ceil((j + 1.0) * bh[:, None]) + y1[:, None], 0.0, HF).astype(jnp.int32)
    len_h = he - hs
    l = jnp.maximum(len_h, 1)
    kh = ((l >= 2).astype(jnp.int32) + (l >= 4).astype(jnp.int32)
          + (l >= 8).astype(jnp.int32))
    row_a = kh * HF + jnp.minimum(hs, HF - 1)
    row_b = kh * HF + jnp.maximum(he - (1 << kh), 0)
    sw16 = jnp.minimum(ws // 16, (WP - KWW) // 16)
    sw = sw16 * 16
    bins = jnp.concatenate([row_a, row_b, len_h, sw16, ws - sw, we - sw], axis=1)
    bins = jnp.pad(bins, ((0, NPAD - NPROP), (0, 0))).reshape(NPAD, 1, 6 * P)

    fhwc = feat[0].transpose(1, 2, 0).astype(jnp.bfloat16)

    pooled = pl.pallas_call(
        _pool_kernel,
        grid=(NPAD // RB,),
        in_specs=[
            pl.BlockSpec((RB, 1, 6 * P), lambda i: (i, 0, 0),
                         memory_space=pltpu.SMEM),
            pl.BlockSpec((HF, WF, C), lambda i: (0, 0, 0)),
        ],
        out_specs=pl.BlockSpec((RB, P * P, C), lambda i: (i, 0, 0)),
        out_shape=jax.ShapeDtypeStruct((NPAD, P * P, C), jnp.bfloat16),
        scratch_shapes=[pltpu.VMEM((P, WP, C), jnp.bfloat16),
                        pltpu.VMEM((4 * HF, WF, C), jnp.bfloat16)],
    )(bins, fhwc)
    x = pooled.reshape(NPAD, KDIM)

    w6q = W6.astype(jnp.bfloat16).reshape(FC, C, P, P).transpose(0, 3, 2, 1).reshape(FC, KDIM)
    wb_perm = Wb.reshape(NC, 4, FC).transpose(1, 0, 2).reshape(4 * NC, FC)
    wh = jnp.concatenate([Wc, wb_perm], axis=0).astype(jnp.bfloat16)
    bhv = jnp.concatenate([bc, bb.reshape(NC, 4).T.reshape(-1)])[None, :]
    prop_p = jnp.pad(proposals, ((0, NPAD - NPROP), (0, 0)))

    outs = pl.pallas_call(
        _fc_kernel,
        grid=(NPAD // BM, KT),
        in_specs=[
            pl.BlockSpec((BM, BK), lambda m, k: (m, k)),
            pl.BlockSpec((FC, BK), lambda m, k: (0, k)),
            pl.BlockSpec((1, FC), lambda m, k: (0, 0)),
            pl.BlockSpec((FC, FC), lambda m, k: (0, 0)),
            pl.BlockSpec((1, FC), lambda m, k: (0, 0)),
            pl.BlockSpec((5 * NC, FC), lambda m, k: (0, 0)),
            pl.BlockSpec((1, 5 * NC), lambda m, k: (0, 0)),
            pl.BlockSpec((BM, 4), lambda m, k: (m, 0)),
        ],
        out_specs=[pl.BlockSpec((BM, NC), lambda m, k: (m, 0))] * 5,
        out_shape=[jax.ShapeDtypeStruct((NPAD, NC), jnp.float32)] * 5,
        scratch_shapes=[pltpu.VMEM((BM, FC), jnp.float32)],
    )(x, w6q, b6[None, :], W7.astype(jnp.bfloat16), b7[None, :], wh, bhv,
      prop_p)
    sc, x1o, y1o, x2o, y2o = outs

    pred_boxes = jnp.stack(
        [x1o[:NPROP], y1o[:NPROP], x2o[:NPROP], y2o[:NPROP]], axis=2)
    pred_scores = sc[:NPROP]
    return pred_boxes, pred_scores


# speedup vs baseline: 1.1070x; 1.1070x over previous
import math

import jax
import jax.numpy as jnp
from jax import lax
from jax.experimental import pallas as pl
from jax.experimental.pallas import tpu as pltpu

C = 256
HF = 50
WF = 50
P = 7
KW = 12
RB = 8
TB = 64
KWW = 32
WP = 64
NPROP = 1000
NPAD = 1024
NC = 91
FC = 1024
KDIM = C * P * P
BM = 256
KT = 7
BK = KDIM // KT
NEG = -1e30
BBOX_CLIP = math.log(1000.0 / 16)


def _pool_kernel(b_ref, f_ref, o_ref, out1_ref, pyr_ref):
    @pl.when(pl.program_id(0) == 0)
    def _build():
        def cp(h, _):
            pyr_ref[pl.ds(h, 1), :, :] = f_ref[pl.ds(h, 1), :, :]
            return 0
        lax.fori_loop(0, HF, cp, 0)
        for k in (1, 2, 3):
            d = 1 << (k - 1)

            def bd(h, _, base=k * HF, pbase=(k - 1) * HF, d=d):
                a = pyr_ref[pl.ds(pbase + h, 1), :, :]
                b = pyr_ref[pl.ds(pbase + jnp.minimum(h + d, HF - 1), 1), :, :]
                pyr_ref[pl.ds(base + h, 1), :, :] = jnp.maximum(a, b)
                return 0

            lax.fori_loop(0, HF, bd, 0)

    def roi_body(i, _):
        for ph in range(P):
            ra = b_ref[i, 0, ph]
            rb = b_ref[i, 0, P + ph]
            ln = b_ref[i, 0, 2 * P + ph]
            a = pyr_ref[pl.ds(ra, 1), :, :]
            b = pyr_ref[pl.ds(rb, 1), :, :]
            v = jnp.maximum(a, b)[0]
            v = jnp.where(ln > 0, v, NEG)
            out1_ref[ph, 0:WF, :] = v
        for pw in range(P):
            sw = b_ref[i, 0, 3 * P + pw] * 16
            lo = b_ref[i, 0, 4 * P + pw]
            hi = b_ref[i, 0, 5 * P + pw]
            win = out1_ref[:, pl.ds(sw, KWW), :]
            r = lax.broadcasted_iota(jnp.int32, (1, KWW, 1), 1)
            v = jnp.where((r >= lo) & (r < hi), win, NEG)
            cell = jnp.max(v, axis=1)
            cell = jnp.where(cell < -1e29, 0.0, cell)
            o_ref[pl.ds(i, 1), pw * P:(pw + 1) * P, :] = cell[None]
        return 0

    lax.fori_loop(0, RB, roi_body, 0)


def _xpose_kernel(x_ref, o_ref):
    o_ref[:] = jnp.transpose(x_ref[:], (0, 2, 1))


def _fc_kernel(x_ref, w6_ref, b6_ref, w7_ref, b7_ref, wh_ref, bh_ref, pr_ref,
               sc_ref, x1_ref, y1_ref, x2_ref, y2_ref, acc_ref):
    k = pl.program_id(1)
    dn = (((1,), (1,)), ((), ()))

    @pl.when(k == 0)
    def _():
        acc_ref[:] = jnp.zeros_like(acc_ref)

    acc_ref[:] += lax.dot_general(x_ref[:], w6_ref[:], dn,
                                  preferred_element_type=jnp.float32)

    @pl.when(k == KT - 1)
    def _():
        h6 = jnp.maximum(acc_ref[:] + b6_ref[:], 0.0).astype(jnp.bfloat16)
        h7 = lax.dot_general(h6, w7_ref[:], dn,
                             preferred_element_type=jnp.float32) + b7_ref[:]
        h7 = jnp.maximum(h7, 0.0).astype(jnp.bfloat16)
        head = lax.dot_general(h7, wh_ref[:], dn,
                               preferred_element_type=jnp.float32) + bh_ref[:]
        cls = head[:, 0:NC]
        mx = jnp.max(cls, axis=1, keepdims=True)
        e = jnp.exp(cls - mx)
        sc_ref[:] = e / jnp.sum(e, axis=1, keepdims=True)
        dx = head[:, NC:2 * NC]
        dy = head[:, 2 * NC:3 * NC]
        dw = jnp.minimum(head[:, 3 * NC:4 * NC], BBOX_CLIP)
        dh = jnp.minimum(head[:, 4 * NC:5 * NC], BBOX_CLIP)
        p = pr_ref[:]
        w = p[:, 2:3] - p[:, 0:1]
        h = p[:, 3:4] - p[:, 1:2]
        cx = p[:, 0:1] + w * 0.5
        cy = p[:, 1:2] + h * 0.5
        px = dx * w + cx
        py = dy * h + cy
        pw = jnp.exp(dw) * w
        ph = jnp.exp(dh) * h
        x1_ref[:] = px - 0.5 * pw
        y1_ref[:] = py - 0.5 * ph
        x2_ref[:] = px + 0.5 * pw
        y2_ref[:] = py + 0.5 * ph


def kernel(feat, proposals, image_shape, W6, b6, W7, b7, Wc, bc, Wb, bb):
    fs = jnp.array(feat.shape[-2:], dtype=jnp.float32)
    ms = image_shape.astype(jnp.float32)
    scale = (2.0 ** jnp.round(jnp.log2(fs / ms)))[0]
    r = jnp.floor(proposals * scale + 0.5)
    x1 = r[:, 0]
    y1 = r[:, 1]
    roi_w = jnp.maximum(r[:, 2] - x1 + 1.0, 1.0)
    roi_h = jnp.maximum(r[:, 3] - y1 + 1.0, 1.0)
    bw = roi_w / P
    bh = roi_h / P
    j = jnp.arange(P, dtype=jnp.float32)
    ws = jnp.clip(jnp.floor(j * bw[:, None]) + x1[:, None], 0.0, WF).astype(jnp.int32)
    we = jnp.clip(jnp.ceil((j + 1.0) * bw[:, None]) + x1[:, None], 0.0, WF).astype(jnp.int32)
    hs = jnp.clip(jnp.floor(j * bh[:, None]) + y1[:, None], 0.0, HF).astype(jnp.int32)
    he = jnp.clip(jnp.ceil((j + 1.0) * bh[:, None]) + y1[:, None], 0.0, HF).astype(jnp.int32)
    len_w = we - ws
    l = jnp.maximum(len_w, 1)
    kw = ((l >= 2).astype(jnp.int32) + (l >= 4).astype(jnp.int32)
          + (l >= 8).astype(jnp.int32))
    row_a = kw * WF + jnp.minimum(ws, WF - 1)
    row_b = kw * WF + jnp.maximum(we - (1 << kw), 0)
    sh16 = jnp.minimum(hs // 16, (WP - KWW) // 16)
    sh = sh16 * 16
    bins = jnp.concatenate([row_a, row_b, len_w, sh16, hs - sh, he - sh], axis=1)
    bins = jnp.pad(bins, ((0, NPAD - NPROP), (0, 0))).reshape(NPAD, 1, 6 * P)

    fwhc = feat[0].transpose(2, 1, 0).astype(jnp.bfloat16)

    pooled = pl.pallas_call(
        _pool_kernel,
        grid=(NPAD // RB,),
        in_specs=[
            pl.BlockSpec((RB, 1, 6 * P), lambda i: (i, 0, 0),
                         memory_space=pltpu.SMEM),
            pl.BlockSpec((HF, WF, C), lambda i: (0, 0, 0)),
        ],
        out_specs=pl.BlockSpec((RB, P * P, C), lambda i: (i, 0, 0)),
        out_shape=jax.ShapeDtypeStruct((NPAD, P * P, C), jnp.bfloat16),
        scratch_shapes=[pltpu.VMEM((P, WP, C), jnp.bfloat16),
                        pltpu.VMEM((4 * WF, HF, C), jnp.bfloat16)],
    )(bins, fwhc)

    xc = pl.pallas_call(
        _xpose_kernel,
        grid=(NPAD // TB,),
        in_specs=[pl.BlockSpec((TB, P * P, C), lambda i: (i, 0, 0))],
        out_specs=pl.BlockSpec((TB, C, P * P), lambda i: (i, 0, 0)),
        out_shape=jax.ShapeDtypeStruct((NPAD, C, P * P), jnp.bfloat16),
    )(pooled)
    x = xc.reshape(NPAD, KDIM)

    w6q = W6.astype(jnp.bfloat16)
    wb_perm = Wb.reshape(NC, 4, FC).transpose(1, 0, 2).reshape(4 * NC, FC)
    wh = jnp.concatenate([Wc, wb_perm], axis=0).astype(jnp.bfloat16)
    bhv = jnp.concatenate([bc, bb.reshape(NC, 4).T.reshape(-1)])[None, :]
    prop_p = jnp.pad(proposals, ((0, NPAD - NPROP), (0, 0)))

    outs = pl.pallas_call(
        _fc_kernel,
        grid=(NPAD // BM, KT),
        in_specs=[
            pl.BlockSpec((BM, BK), lambda m, k: (m, k)),
            pl.BlockSpec((FC, BK), lambda m, k: (0, k)),
            pl.BlockSpec((1, FC), lambda m, k: (0, 0)),
            pl.BlockSpec((FC, FC), lambda m, k: (0, 0)),
            pl.BlockSpec((1, FC), lambda m, k: (0, 0)),
            pl.BlockSpec((5 * NC, FC), lambda m, k: (0, 0)),
            pl.BlockSpec((1, 5 * NC), lambda m, k: (0, 0)),
            pl.BlockSpec((BM, 4), lambda m, k: (m, 0)),
        ],
        out_specs=[pl.BlockSpec((BM, NC), lambda m, k: (m, 0))] * 5,
        out_shape=[jax.ShapeDtypeStruct((NPAD, NC), jnp.float32)] * 5,
        scratch_shapes=[pltpu.VMEM((BM, FC), jnp.float32)],
    )(x, w6q, b6[None, :], W7.astype(jnp.bfloat16), b7[None, :], wh, bhv,
      prop_p)
    sc, x1o, y1o, x2o, y2o = outs

    pred_boxes = jnp.stack(
        [x1o[:NPROP], y1o[:NPROP], x2o[:NPROP], y2o[:NPROP]], axis=2)
    pred_scores = sc[:NPROP]
    return pred_boxes, pred_scores


# speedup vs baseline: 1.1438x; 1.0333x over previous
import math

import jax
import jax.numpy as jnp
from jax import lax
from jax.experimental import pallas as pl
from jax.experimental.pallas import tpu as pltpu

C = 256
HF = 50
WF = 50
P = 7
KW = 12
RB = 16
TB = 64
KWW = 32
WP = 64
NPROP = 1000
NPAD = 1024
NC = 91
FC = 1024
KDIM = C * P * P
BM = 256
KT = 7
BK = KDIM // KT
NEG = -1e30
BBOX_CLIP = math.log(1000.0 / 16)


def _pool_kernel(b_ref, f_ref, o_ref, out1_ref, pyr_ref):
    @pl.when(pl.program_id(0) == 0)
    def _build():
        def cp(h, _):
            pyr_ref[pl.ds(h, 1), :, :] = f_ref[pl.ds(h, 1), :, :]
            return 0
        lax.fori_loop(0, HF, cp, 0)
        for k in (1, 2, 3):
            d = 1 << (k - 1)

            def bd(h, _, base=k * HF, pbase=(k - 1) * HF, d=d):
                a = pyr_ref[pl.ds(pbase + h, 1), :, :]
                b = pyr_ref[pl.ds(pbase + jnp.minimum(h + d, HF - 1), 1), :, :]
                pyr_ref[pl.ds(base + h, 1), :, :] = jnp.maximum(a, b)
                return 0

            lax.fori_loop(0, HF, bd, 0)

    def roi_body(i, _):
        for ph in range(P):
            ra = b_ref[i, 0, ph]
            rb = b_ref[i, 0, P + ph]
            ln = b_ref[i, 0, 2 * P + ph]
            a = pyr_ref[pl.ds(ra, 1), :, :]
            b = pyr_ref[pl.ds(rb, 1), :, :]
            v = jnp.maximum(a, b)[0]
            v = jnp.where(ln > 0, v, NEG)
            out1_ref[ph, 0:WF, :] = v
        for pw in range(P):
            sw = b_ref[i, 0, 3 * P + pw] * 16
            lo = b_ref[i, 0, 4 * P + pw]
            hi = b_ref[i, 0, 5 * P + pw]
            win = out1_ref[:, pl.ds(sw, KWW), :]
            r = lax.broadcasted_iota(jnp.int32, (1, KWW, 1), 1)
            v = jnp.where((r >= lo) & (r < hi), win, NEG)
            cell = jnp.max(v, axis=1)
            cell = jnp.where(cell < -1e29, 0.0, cell)
            o_ref[pl.ds(i, 1), pw * P:(pw + 1) * P, :] = cell[None]
        return 0

    lax.fori_loop(0, RB, roi_body, 0)


def _xpose_kernel(x_ref, o_ref):
    o_ref[:] = jnp.transpose(x_ref[:], (0, 2, 1))


def _fc_kernel(x_ref, w6_ref, b6_ref, w7_ref, b7_ref, wh_ref, bh_ref, pr_ref,
               sc_ref, x1_ref, y1_ref, x2_ref, y2_ref):
    dn = (((1,), (1,)), ((), ()))
    h6 = lax.dot_general(x_ref[:], w6_ref[:], dn,
                         preferred_element_type=jnp.float32) + b6_ref[:]
    h6 = jnp.maximum(h6, 0.0).astype(jnp.bfloat16)
    h7 = lax.dot_general(h6, w7_ref[:], dn,
                         preferred_element_type=jnp.float32) + b7_ref[:]
    h7 = jnp.maximum(h7, 0.0).astype(jnp.bfloat16)
    head = lax.dot_general(h7, wh_ref[:], dn,
                           preferred_element_type=jnp.float32) + bh_ref[:]
    cls = head[:, 0:NC]
    mx = jnp.max(cls, axis=1, keepdims=True)
    e = jnp.exp(cls - mx)
    sc_ref[:] = e / jnp.sum(e, axis=1, keepdims=True)
    dx = head[:, NC:2 * NC]
    dy = head[:, 2 * NC:3 * NC]
    dw = jnp.minimum(head[:, 3 * NC:4 * NC], BBOX_CLIP)
    dh = jnp.minimum(head[:, 4 * NC:5 * NC], BBOX_CLIP)
    p = pr_ref[:]
    w = p[:, 2:3] - p[:, 0:1]
    h = p[:, 3:4] - p[:, 1:2]
    cx = p[:, 0:1] + w * 0.5
    cy = p[:, 1:2] + h * 0.5
    px = dx * w + cx
    py = dy * h + cy
    pw = jnp.exp(dw) * w
    ph = jnp.exp(dh) * h
    x1_ref[:] = px - 0.5 * pw
    y1_ref[:] = py - 0.5 * ph
    x2_ref[:] = px + 0.5 * pw
    y2_ref[:] = py + 0.5 * ph


def kernel(feat, proposals, image_shape, W6, b6, W7, b7, Wc, bc, Wb, bb):
    fs = jnp.array(feat.shape[-2:], dtype=jnp.float32)
    ms = image_shape.astype(jnp.float32)
    scale = (2.0 ** jnp.round(jnp.log2(fs / ms)))[0]
    r = jnp.floor(proposals * scale + 0.5)
    x1 = r[:, 0]
    y1 = r[:, 1]
    roi_w = jnp.maximum(r[:, 2] - x1 + 1.0, 1.0)
    roi_h = jnp.maximum(r[:, 3] - y1 + 1.0, 1.0)
    bw = roi_w / P
    bh = roi_h / P
    j = jnp.arange(P, dtype=jnp.float32)
    ws = jnp.clip(jnp.floor(j * bw[:, None]) + x1[:, None], 0.0, WF).astype(jnp.int32)
    we = jnp.clip(jnp.ceil((j + 1.0) * bw[:, None]) + x1[:, None], 0.0, WF).astype(jnp.int32)
    hs = jnp.clip(jnp.floor(j * bh[:, None]) + y1[:, None], 0.0, HF).astype(jnp.int32)
    he = jnp.clip(jnp.ceil((j + 1.0) * bh[:, None]) + y1[:, None], 0.0, HF).astype(jnp.int32)
    len_w = we - ws
    l = jnp.maximum(len_w, 1)
    kw = ((l >= 2).astype(jnp.int32) + (l >= 4).astype(jnp.int32)
          + (l >= 8).astype(jnp.int32))
    row_a = kw * WF + jnp.minimum(ws, WF - 1)
    row_b = kw * WF + jnp.maximum(we - (1 << kw), 0)
    sh16 = jnp.minimum(hs // 16, (WP - KWW) // 16)
    sh = sh16 * 16
    bins = jnp.concatenate([row_a, row_b, len_w, sh16, hs - sh, he - sh], axis=1)
    bins = jnp.pad(bins, ((0, NPAD - NPROP), (0, 0))).reshape(NPAD, 1, 6 * P)

    fwhc = feat[0].transpose(2, 1, 0).astype(jnp.bfloat16)

    pooled = pl.pallas_call(
        _pool_kernel,
        grid=(NPAD // RB,),
        in_specs=[
            pl.BlockSpec((RB, 1, 6 * P), lambda i: (i, 0, 0),
                         memory_space=pltpu.SMEM),
            pl.BlockSpec((HF, WF, C), lambda i: (0, 0, 0)),
        ],
        out_specs=pl.BlockSpec((RB, P * P, C), lambda i: (i, 0, 0)),
        out_shape=jax.ShapeDtypeStruct((NPAD, P * P, C), jnp.bfloat16),
        scratch_shapes=[pltpu.VMEM((P, WP, C), jnp.bfloat16),
                        pltpu.VMEM((4 * WF, HF, C), jnp.bfloat16)],
    )(bins, fwhc)

    xc = pl.pallas_call(
        _xpose_kernel,
        grid=(NPAD // TB,),
        in_specs=[pl.BlockSpec((TB, P * P, C), lambda i: (i, 0, 0))],
        out_specs=pl.BlockSpec((TB, C, P * P), lambda i: (i, 0, 0)),
        out_shape=jax.ShapeDtypeStruct((NPAD, C, P * P), jnp.bfloat16),
    )(pooled)
    x = xc.reshape(NPAD, KDIM)

    w6q = W6.astype(jnp.bfloat16)
    wb_perm = Wb.reshape(NC, 4, FC).transpose(1, 0, 2).reshape(4 * NC, FC)
    wh = jnp.concatenate([Wc, wb_perm], axis=0).astype(jnp.bfloat16)
    bhv = jnp.concatenate([bc, bb.reshape(NC, 4).T.reshape(-1)])[None, :]
    prop_p = jnp.pad(proposals, ((0, NPAD - NPROP), (0, 0)))

    outs = pl.pallas_call(
        _fc_kernel,
        grid=(NPAD // BM,),
        in_specs=[
            pl.BlockSpec((BM, KDIM), lambda m: (m, 0)),
            pl.BlockSpec((FC, KDIM), lambda m: (0, 0)),
            pl.BlockSpec((1, FC), lambda m: (0, 0)),
            pl.BlockSpec((FC, FC), lambda m: (0, 0)),
            pl.BlockSpec((1, FC), lambda m: (0, 0)),
            pl.BlockSpec((5 * NC, FC), lambda m: (0, 0)),
            pl.BlockSpec((1, 5 * NC), lambda m: (0, 0)),
            pl.BlockSpec((BM, 4), lambda m: (m, 0)),
        ],
        out_specs=[pl.BlockSpec((BM, NC), lambda m: (m, 0))] * 5,
        out_shape=[jax.ShapeDtypeStruct((NPAD, NC), jnp.float32)] * 5,
    )(x, w6q, b6[None, :], W7.astype(jnp.bfloat16), b7[None, :], wh, bhv,
      prop_p)
    sc, x1o, y1o, x2o, y2o = outs

    pred_boxes = jnp.stack(
        [x1o[:NPROP], y1o[:NPROP], x2o[:NPROP], y2o[:NPROP]], axis=2)
    pred_scores = sc[:NPROP]
    return pred_boxes, pred_scores


# speedup vs baseline: 1.2521x; 1.0946x over previous
import math

import jax
import jax.numpy as jnp
from jax import lax
from jax.experimental import pallas as pl
from jax.experimental.pallas import tpu as pltpu

C = 256
HF = 50
WF = 50
P = 7
KW = 12
RB = 16
TB = 64
KWW = 16
WP = 128
NPROP = 1000
NPAD = 1024
NC = 91
FC = 1024
KDIM = C * P * P
BM = 256
KT = 7
BK = KDIM // KT
NEG = -1e30
BBOX_CLIP = math.log(1000.0 / 16)


def _pool_kernel(b_ref, f_ref, o_ref, out1_ref, pyr_ref):
    @pl.when(pl.program_id(0) == 0)
    def _build():
        def cp(h, _):
            pyr_ref[pl.ds(h, 1), :, :] = f_ref[pl.ds(h, 1), :, :]
            return 0
        lax.fori_loop(0, HF, cp, 0)
        for k in (1, 2, 3):
            d = 1 << (k - 1)

            def bd(h, _, base=k * HF, pbase=(k - 1) * HF, d=d):
                a = pyr_ref[pl.ds(pbase + h, 1), :, :]
                b = pyr_ref[pl.ds(pbase + jnp.minimum(h + d, HF - 1), 1), :, :]
                pyr_ref[pl.ds(base + h, 1), :, :] = jnp.maximum(a, b)
                return 0

            lax.fori_loop(0, HF, bd, 0)

    def roi_body(i, _):
        for ph in range(P):
            ra = b_ref[i, 0, ph]
            rb = b_ref[i, 0, P + ph]
            ln = b_ref[i, 0, 2 * P + ph]
            a = pyr_ref[pl.ds(ra, 1), :, :]
            b = pyr_ref[pl.ds(rb, 1), :, :]
            v = jnp.maximum(a, b)[0]
            v = jnp.where(ln > 0, v, NEG)
            out1_ref[ph, 0:HF, :] = v
            out1_ref[ph, 56:56 + HF, :] = v
        for pw in range(P):
            sw = b_ref[i, 0, 3 * P + pw] * 16
            lo = b_ref[i, 0, 4 * P + pw]
            hi = b_ref[i, 0, 5 * P + pw]
            win = out1_ref[:, pl.ds(sw, KWW), :]
            r = lax.broadcasted_iota(jnp.int32, (1, KWW, 1), 1)
            v = jnp.where((r >= lo) & (r < hi), win, NEG)
            cell = jnp.max(v, axis=1)
            cell = jnp.where(cell < -1e29, 0.0, cell)
            o_ref[pl.ds(i, 1), pw * P:(pw + 1) * P, :] = cell[None]
        return 0

    lax.fori_loop(0, RB, roi_body, 0)


def _xpose_kernel(x_ref, o_ref):
    o_ref[:] = jnp.transpose(x_ref[:], (0, 2, 1))


def _fc_kernel(x_ref, w6_ref, b6_ref, w7_ref, b7_ref, wh_ref, bh_ref, pr_ref,
               sc_ref, x1_ref, y1_ref, x2_ref, y2_ref):
    dn = (((1,), (1,)), ((), ()))
    h6 = lax.dot_general(x_ref[:], w6_ref[:], dn,
                         preferred_element_type=jnp.float32) + b6_ref[:]
    h6 = jnp.maximum(h6, 0.0).astype(jnp.bfloat16)
    h7 = lax.dot_general(h6, w7_ref[:], dn,
                         preferred_element_type=jnp.float32) + b7_ref[:]
    h7 = jnp.maximum(h7, 0.0).astype(jnp.bfloat16)
    head = lax.dot_general(h7, wh_ref[:], dn,
                           preferred_element_type=jnp.float32) + bh_ref[:]
    cls = head[:, 0:NC]
    mx = jnp.max(cls, axis=1, keepdims=True)
    e = jnp.exp(cls - mx)
    sc_ref[:] = e / jnp.sum(e, axis=1, keepdims=True)
    dx = head[:, NC:2 * NC]
    dy = head[:, 2 * NC:3 * NC]
    dw = jnp.minimum(head[:, 3 * NC:4 * NC], BBOX_CLIP)
    dh = jnp.minimum(head[:, 4 * NC:5 * NC], BBOX_CLIP)
    p = pr_ref[:]
    w = p[:, 2:3] - p[:, 0:1]
    h = p[:, 3:4] - p[:, 1:2]
    cx = p[:, 0:1] + w * 0.5
    cy = p[:, 1:2] + h * 0.5
    px = dx * w + cx
    py = dy * h + cy
    pw = jnp.exp(dw) * w
    ph = jnp.exp(dh) * h
    x1_ref[:] = px - 0.5 * pw
    y1_ref[:] = py - 0.5 * ph
    x2_ref[:] = px + 0.5 * pw
    y2_ref[:] = py + 0.5 * ph


def kernel(feat, proposals, image_shape, W6, b6, W7, b7, Wc, bc, Wb, bb):
    fs = jnp.array(feat.shape[-2:], dtype=jnp.float32)
    ms = image_shape.astype(jnp.float32)
    scale = (2.0 ** jnp.round(jnp.log2(fs / ms)))[0]
    r = jnp.floor(proposals * scale + 0.5)
    x1 = r[:, 0]
    y1 = r[:, 1]
    roi_w = jnp.maximum(r[:, 2] - x1 + 1.0, 1.0)
    roi_h = jnp.maximum(r[:, 3] - y1 + 1.0, 1.0)
    bw = roi_w / P
    bh = roi_h / P
    j = jnp.arange(P, dtype=jnp.float32)
    ws = jnp.clip(jnp.floor(j * bw[:, None]) + x1[:, None], 0.0, WF).astype(jnp.int32)
    we = jnp.clip(jnp.ceil((j + 1.0) * bw[:, None]) + x1[:, None], 0.0, WF).astype(jnp.int32)
    hs = jnp.clip(jnp.floor(j * bh[:, None]) + y1[:, None], 0.0, HF).astype(jnp.int32)
    he = jnp.clip(jnp.ceil((j + 1.0) * bh[:, None]) + y1[:, None], 0.0, HF).astype(jnp.int32)
    len_w = we - ws
    l = jnp.maximum(len_w, 1)
    kw = ((l >= 2).astype(jnp.int32) + (l >= 4).astype(jnp.int32)
          + (l >= 8).astype(jnp.int32))
    row_a = kw * WF + jnp.minimum(ws, WF - 1)
    row_b = kw * WF + jnp.maximum(we - (1 << kw), 0)
    pos = jnp.where(hs % 16 >= 8, hs + 56, hs)
    sh16 = pos // 16
    lo_h = pos % 16
    bins = jnp.concatenate([row_a, row_b, len_w, sh16, lo_h, lo_h + he - hs],
                           axis=1)
    bins = jnp.pad(bins, ((0, NPAD - NPROP), (0, 0))).reshape(NPAD, 1, 6 * P)

    fwhc = feat[0].transpose(2, 1, 0).astype(jnp.bfloat16)

    pooled = pl.pallas_call(
        _pool_kernel,
        grid=(NPAD // RB,),
        in_specs=[
            pl.BlockSpec((RB, 1, 6 * P), lambda i: (i, 0, 0),
                         memory_space=pltpu.SMEM),
            pl.BlockSpec((HF, WF, C), lambda i: (0, 0, 0)),
        ],
        out_specs=pl.BlockSpec((RB, P * P, C), lambda i: (i, 0, 0)),
        out_shape=jax.ShapeDtypeStruct((NPAD, P * P, C), jnp.bfloat16),
        scratch_shapes=[pltpu.VMEM((P, WP, C), jnp.bfloat16),
                        pltpu.VMEM((4 * WF, HF, C), jnp.bfloat16)],
    )(bins, fwhc)

    xc = pl.pallas_call(
        _xpose_kernel,
        grid=(NPAD // TB,),
        in_specs=[pl.BlockSpec((TB, P * P, C), lambda i: (i, 0, 0))],
        out_specs=pl.BlockSpec((TB, C, P * P), lambda i: (i, 0, 0)),
        out_shape=jax.ShapeDtypeStruct((NPAD, C, P * P), jnp.bfloat16),
    )(pooled)
    x = xc.reshape(NPAD, KDIM)

    w6q = W6.astype(jnp.bfloat16)
    wb_perm = Wb.reshape(NC, 4, FC).transpose(1, 0, 2).reshape(4 * NC, FC)
    wh = jnp.concatenate([Wc, wb_perm], axis=0).astype(jnp.bfloat16)
    bhv = jnp.concatenate([bc, bb.reshape(NC, 4).T.reshape(-1)])[None, :]
    prop_p = jnp.pad(proposals, ((0, NPAD - NPROP), (0, 0)))

    outs = pl.pallas_call(
        _fc_kernel,
        grid=(NPAD // BM,),
        in_specs=[
            pl.BlockSpec((BM, KDIM), lambda m: (m, 0)),
            pl.BlockSpec((FC, KDIM), lambda m: (0, 0)),
            pl.BlockSpec((1, FC), lambda m: (0, 0)),
            pl.BlockSpec((FC, FC), lambda m: (0, 0)),
            pl.BlockSpec((1, FC), lambda m: (0, 0)),
            pl.BlockSpec((5 * NC, FC), lambda m: (0, 0)),
            pl.BlockSpec((1, 5 * NC), lambda m: (0, 0)),
            pl.BlockSpec((BM, 4), lambda m: (m, 0)),
        ],
        out_specs=[pl.BlockSpec((BM, NC), lambda m: (m, 0))] * 5,
        out_shape=[jax.ShapeDtypeStruct((NPAD, NC), jnp.float32)] * 5,
    )(x, w6q, b6[None, :], W7.astype(jnp.bfloat16), b7[None, :], wh, bhv,
      prop_p)
    sc, x1o, y1o, x2o, y2o = outs

    pred_boxes = jnp.stack(
        [x1o[:NPROP], y1o[:NPROP], x2o[:NPROP], y2o[:NPROP]], axis=2)
    pred_scores = sc[:NPROP]
    return pred_boxes, pred_scores


# speedup vs baseline: 1.2607x; 1.0069x over previous
import math

import jax
import jax.numpy as jnp
from jax import lax
from jax.experimental import pallas as pl
from jax.experimental.pallas import tpu as pltpu

C = 256
HF = 50
WF = 50
P = 7
KW = 12
RB = 16
TB = 64
KWW = 16
WP = 128
NPROP = 1000
NPAD = 1024
NC = 91
FC = 1024
KDIM = C * P * P
BM = 256
KT = 7
BK = KDIM // KT
NEG = -1e30
BBOX_CLIP = math.log(1000.0 / 16)


def _pool_kernel(b_ref, f_ref, o_ref, out1_ref, pyr_ref):
    @pl.when(pl.program_id(0) == 0)
    def _build():
        def cp(h, _):
            pyr_ref[pl.ds(h, 1), :, :] = f_ref[pl.ds(h, 1), :, :]
            return 0
        lax.fori_loop(0, HF, cp, 0)
        for k in (1, 2, 3):
            d = 1 << (k - 1)

            def bd(h, _, base=k * HF, pbase=(k - 1) * HF, d=d):
                a = pyr_ref[pl.ds(pbase + h, 1), :, :]
                b = pyr_ref[pl.ds(pbase + jnp.minimum(h + d, HF - 1), 1), :, :]
                pyr_ref[pl.ds(base + h, 1), :, :] = jnp.maximum(a, b)
                return 0

            lax.fori_loop(0, HF, bd, 0)

    def roi_body(i, _):
        for ph in range(P):
            ra = b_ref[i, 0, ph]
            rb = b_ref[i, 0, P + ph]
            ln = b_ref[i, 0, 2 * P + ph]
            a = pyr_ref[pl.ds(ra, 1), :, :]
            b = pyr_ref[pl.ds(rb, 1), :, :]
            v = jnp.maximum(a, b)[0]
            v = jnp.where(ln > 0, v, NEG)
            out1_ref[ph, 0:HF, :] = v
            out1_ref[ph, 56:56 + HF, :] = v
        for pw in range(P):
            sw = b_ref[i, 0, 3 * P + pw] * 16
            lo = b_ref[i, 0, 4 * P + pw]
            hi = b_ref[i, 0, 5 * P + pw]
            win = out1_ref[:, pl.ds(sw, KWW), :]
            r = lax.broadcasted_iota(jnp.int32, (1, KWW, 1), 1)
            v = jnp.where((r >= lo) & (r < hi), win, NEG)
            cell = jnp.max(v, axis=1)
            cell = jnp.where(cell < -1e29, 0.0, cell)
            o_ref[pl.ds(i, 1), pw * P:(pw + 1) * P, :] = cell[None]
        return 0

    lax.fori_loop(0, RB, roi_body, 0)


def _xpose_kernel(x_ref, w_ref, o_ref, ow_ref):
    o_ref[:] = jnp.transpose(x_ref[:], (0, 2, 1))
    ow_ref[:] = w_ref[:].astype(jnp.bfloat16)


def _fc_kernel(x_ref, w6_ref, b6_ref, w7_ref, b7_ref, wh_ref, bh_ref, pr_ref,
               sc_ref, x1_ref, y1_ref, x2_ref, y2_ref):
    dn = (((1,), (1,)), ((), ()))
    h6 = lax.dot_general(x_ref[:], w6_ref[:], dn,
                         preferred_element_type=jnp.float32) + b6_ref[:]
    h6 = jnp.maximum(h6, 0.0).astype(jnp.bfloat16)
    h7 = lax.dot_general(h6, w7_ref[:], dn,
                         preferred_element_type=jnp.float32) + b7_ref[:]
    h7 = jnp.maximum(h7, 0.0).astype(jnp.bfloat16)
    head = lax.dot_general(h7, wh_ref[:], dn,
                           preferred_element_type=jnp.float32) + bh_ref[:]
    cls = head[:, 0:NC]
    mx = jnp.max(cls, axis=1, keepdims=True)
    e = jnp.exp(cls - mx)
    sc_ref[:] = e / jnp.sum(e, axis=1, keepdims=True)
    dx = head[:, NC:2 * NC]
    dy = head[:, 2 * NC:3 * NC]
    dw = jnp.minimum(head[:, 3 * NC:4 * NC], BBOX_CLIP)
    dh = jnp.minimum(head[:, 4 * NC:5 * NC], BBOX_CLIP)
    p = pr_ref[:]
    w = p[:, 2:3] - p[:, 0:1]
    h = p[:, 3:4] - p[:, 1:2]
    cx = p[:, 0:1] + w * 0.5
    cy = p[:, 1:2] + h * 0.5
    px = dx * w + cx
    py = dy * h + cy
    pw = jnp.exp(dw) * w
    ph = jnp.exp(dh) * h
    x1_ref[:] = px - 0.5 * pw
    y1_ref[:] = py - 0.5 * ph
    x2_ref[:] = px + 0.5 * pw
    y2_ref[:] = py + 0.5 * ph


def kernel(feat, proposals, image_shape, W6, b6, W7, b7, Wc, bc, Wb, bb):
    fs = jnp.array(feat.shape[-2:], dtype=jnp.float32)
    ms = image_shape.astype(jnp.float32)
    scale = (2.0 ** jnp.round(jnp.log2(fs / ms)))[0]
    r = jnp.floor(proposals * scale + 0.5)
    x1 = r[:, 0]
    y1 = r[:, 1]
    roi_w = jnp.maximum(r[:, 2] - x1 + 1.0, 1.0)
    roi_h = jnp.maximum(r[:, 3] - y1 + 1.0, 1.0)
    bw = roi_w / P
    bh = roi_h / P
    j = jnp.arange(P, dtype=jnp.float32)
    ws = jnp.clip(jnp.floor(j * bw[:, None]) + x1[:, None], 0.0, WF).astype(jnp.int32)
    we = jnp.clip(jnp.ceil((j + 1.0) * bw[:, None]) + x1[:, None], 0.0, WF).astype(jnp.int32)
    hs = jnp.clip(jnp.floor(j * bh[:, None]) + y1[:, None], 0.0, HF).astype(jnp.int32)
    he = jnp.clip(jnp.ceil((j + 1.0) * bh[:, None]) + y1[:, None], 0.0, HF).astype(jnp.int32)
    len_w = we - ws
    l = jnp.maximum(len_w, 1)
    kw = ((l >= 2).astype(jnp.int32) + (l >= 4).astype(jnp.int32)
          + (l >= 8).astype(jnp.int32))
    row_a = kw * WF + jnp.minimum(ws, WF - 1)
    row_b = kw * WF + jnp.maximum(we - (1 << kw), 0)
    pos = jnp.where(hs % 16 >= 8, hs + 56, hs)
    sh16 = pos // 16
    lo_h = pos % 16
    bins = jnp.concatenate([row_a, row_b, len_w, sh16, lo_h, lo_h + he - hs],
                           axis=1)
    bins = jnp.pad(bins, ((0, NPAD - NPROP), (0, 0))).reshape(NPAD, 1, 6 * P)

    fwhc = feat[0].transpose(2, 1, 0).astype(jnp.bfloat16)

    pooled = pl.pallas_call(
        _pool_kernel,
        grid=(NPAD // RB,),
        in_specs=[
            pl.BlockSpec((RB, 1, 6 * P), lambda i: (i, 0, 0),
                         memory_space=pltpu.SMEM),
            pl.BlockSpec((HF, WF, C), lambda i: (0, 0, 0)),
        ],
        out_specs=pl.BlockSpec((RB, P * P, C), lambda i: (i, 0, 0)),
        out_shape=jax.ShapeDtypeStruct((NPAD, P * P, C), jnp.bfloat16),
        scratch_shapes=[pltpu.VMEM((P, WP, C), jnp.bfloat16),
                        pltpu.VMEM((4 * WF, HF, C), jnp.bfloat16)],
    )(bins, fwhc)

    xc, w6q = pl.pallas_call(
        _xpose_kernel,
        grid=(NPAD // TB,),
        in_specs=[pl.BlockSpec((TB, P * P, C), lambda i: (i, 0, 0)),
                  pl.BlockSpec((TB, KDIM), lambda i: (i, 0))],
        out_specs=[pl.BlockSpec((TB, C, P * P), lambda i: (i, 0, 0)),
                   pl.BlockSpec((TB, KDIM), lambda i: (i, 0))],
        out_shape=[jax.ShapeDtypeStruct((NPAD, C, P * P), jnp.bfloat16),
                   jax.ShapeDtypeStruct((FC, KDIM), jnp.bfloat16)],
    )(pooled, W6)
    x = xc.reshape(NPAD, KDIM)

    wb_perm = Wb.reshape(NC, 4, FC).transpose(1, 0, 2).reshape(4 * NC, FC)
    wh = jnp.concatenate([Wc, wb_perm], axis=0).astype(jnp.bfloat16)
    bhv = jnp.concatenate([bc, bb.reshape(NC, 4).T.reshape(-1)])[None, :]
    prop_p = jnp.pad(proposals, ((0, NPAD - NPROP), (0, 0)))

    outs = pl.pallas_call(
        _fc_kernel,
        grid=(NPAD // BM,),
        in_specs=[
            pl.BlockSpec((BM, KDIM), lambda m: (m, 0)),
            pl.BlockSpec((FC, KDIM), lambda m: (0, 0)),
            pl.BlockSpec((1, FC), lambda m: (0, 0)),
            pl.BlockSpec((FC, FC), lambda m: (0, 0)),
            pl.BlockSpec((1, FC), lambda m: (0, 0)),
            pl.BlockSpec((5 * NC, FC), lambda m: (0, 0)),
            pl.BlockSpec((1, 5 * NC), lambda m: (0, 0)),
            pl.BlockSpec((BM, 4), lambda m: (m, 0)),
        ],
        out_specs=[pl.BlockSpec((BM, NC), lambda m: (m, 0))] * 5,
        out_shape=[jax.ShapeDtypeStruct((NPAD, NC), jnp.float32)] * 5,
    )(x, w6q, b6[None, :], W7.astype(jnp.bfloat16), b7[None, :], wh, bhv,
      prop_p)
    sc, x1o, y1o, x2o, y2o = outs

    pred_boxes = jnp.stack(
        [x1o[:NPROP], y1o[:NPROP], x2o[:NPROP], y2o[:NPROP]], axis=2)
    pred_scores = sc[:NPROP]
    return pred_boxes, pred_scores


# speedup vs baseline: 1.2635x; 1.0022x over previous
import math

import jax
import jax.numpy as jnp
from jax import lax
from jax.experimental import pallas as pl
from jax.experimental.pallas import tpu as pltpu

C = 256
HF = 50
WF = 50
P = 7
KW = 12
RB = 32
TB = 64
KWW = 16
WP = 128
NPROP = 1000
NPAD = 1024
NC = 91
FC = 1024
KDIM = C * P * P
BM = 256
KT = 7
BK = KDIM // KT
NEG = -1e30
BBOX_CLIP = math.log(1000.0 / 16)


def _pool_kernel(b_ref, f_ref, o_ref, out1_ref, pyr_ref):
    @pl.when(pl.program_id(0) == 0)
    def _build():
        def cp(h, _):
            pyr_ref[pl.ds(h, 1), :, :] = f_ref[pl.ds(h, 1), :, :]
            return 0
        lax.fori_loop(0, HF, cp, 0)
        for k in (1, 2, 3):
            d = 1 << (k - 1)

            def bd(h, _, base=k * HF, pbase=(k - 1) * HF, d=d):
                a = pyr_ref[pl.ds(pbase + h, 1), :, :]
                b = pyr_ref[pl.ds(pbase + jnp.minimum(h + d, HF - 1), 1), :, :]
                pyr_ref[pl.ds(base + h, 1), :, :] = jnp.maximum(a, b)
                return 0

            lax.fori_loop(0, HF, bd, 0)

    def roi_body(i, _):
        for ph in range(P):
            ra = b_ref[i, 0, ph]
            rb = b_ref[i, 0, P + ph]
            ln = b_ref[i, 0, 2 * P + ph]
            a = pyr_ref[pl.ds(ra, 1), :, :]
            b = pyr_ref[pl.ds(rb, 1), :, :]
            v = jnp.maximum(a, b)[0]
            v = jnp.where(ln > 0, v, NEG)
            out1_ref[ph, 0:HF, :] = v
            out1_ref[ph, 56:56 + HF, :] = v
        for pw in range(P):
            sw = b_ref[i, 0, 3 * P + pw] * 16
            lo = b_ref[i, 0, 4 * P + pw]
            hi = b_ref[i, 0, 5 * P + pw]
            win = out1_ref[:, pl.ds(sw, KWW), :]
            r = lax.broadcasted_iota(jnp.int32, (1, KWW, 1), 1)
            v = jnp.where((r >= lo) & (r < hi), win, NEG)
            cell = jnp.max(v, axis=1)
            cell = jnp.where(cell < -1e29, 0.0, cell)
            o_ref[pl.ds(i, 1), pw * P:(pw + 1) * P, :] = cell[None]
        return 0

    lax.fori_loop(0, RB, roi_body, 0)


def _xpose_kernel(x_ref, w_ref, o_ref, ow_ref):
    o_ref[:] = jnp.transpose(x_ref[:], (0, 2, 1))
    ow_ref[:] = w_ref[:].astype(jnp.bfloat16)


def _fc_kernel(x_ref, w6_ref, b6_ref, w7_ref, b7_ref, wh_ref, bh_ref, pr_ref,
               sc_ref, x1_ref, y1_ref, x2_ref, y2_ref):
    dn = (((1,), (1,)), ((), ()))
    h6 = lax.dot_general(x_ref[:], w6_ref[:], dn,
                         preferred_element_type=jnp.float32) + b6_ref[:]
    h6 = jnp.maximum(h6, 0.0).astype(jnp.bfloat16)
    h7 = lax.dot_general(h6, w7_ref[:], dn,
                         preferred_element_type=jnp.float32) + b7_ref[:]
    h7 = jnp.maximum(h7, 0.0).astype(jnp.bfloat16)
    head = lax.dot_general(h7, wh_ref[:], dn,
                           preferred_element_type=jnp.float32) + bh_ref[:]
    cls = head[:, 0:NC]
    mx = jnp.max(cls, axis=1, keepdims=True)
    e = jnp.exp(cls - mx)
    sc_ref[:] = e / jnp.sum(e, axis=1, keepdims=True)
    dx = head[:, NC:2 * NC]
    dy = head[:, 2 * NC:3 * NC]
    dw = jnp.minimum(head[:, 3 * NC:4 * NC], BBOX_CLIP)
    dh = jnp.minimum(head[:, 4 * NC:5 * NC], BBOX_CLIP)
    p = pr_ref[:]
    w = p[:, 2:3] - p[:, 0:1]
    h = p[:, 3:4] - p[:, 1:2]
    cx = p[:, 0:1] + w * 0.5
    cy = p[:, 1:2] + h * 0.5
    px = dx * w + cx
    py = dy * h + cy
    pw = jnp.exp(dw) * w
    ph = jnp.exp(dh) * h
    x1_ref[:] = px - 0.5 * pw
    y1_ref[:] = py - 0.5 * ph
    x2_ref[:] = px + 0.5 * pw
    y2_ref[:] = py + 0.5 * ph


def kernel(feat, proposals, image_shape, W6, b6, W7, b7, Wc, bc, Wb, bb):
    fs = jnp.array(feat.shape[-2:], dtype=jnp.float32)
    ms = image_shape.astype(jnp.float32)
    scale = (2.0 ** jnp.round(jnp.log2(fs / ms)))[0]
    r = jnp.floor(proposals * scale + 0.5)
    x1 = r[:, 0]
    y1 = r[:, 1]
    roi_w = jnp.maximum(r[:, 2] - x1 + 1.0, 1.0)
    roi_h = jnp.maximum(r[:, 3] - y1 + 1.0, 1.0)
    bw = roi_w / P
    bh = roi_h / P
    j = jnp.arange(P, dtype=jnp.float32)
    ws = jnp.clip(jnp.floor(j * bw[:, None]) + x1[:, None], 0.0, WF).astype(jnp.int32)
    we = jnp.clip(jnp.ceil((j + 1.0) * bw[:, None]) + x1[:, None], 0.0, WF).astype(jnp.int32)
    hs = jnp.clip(jnp.floor(j * bh[:, None]) + y1[:, None], 0.0, HF).astype(jnp.int32)
    he = jnp.clip(jnp.ceil((j + 1.0) * bh[:, None]) + y1[:, None], 0.0, HF).astype(jnp.int32)
    len_w = we - ws
    l = jnp.maximum(len_w, 1)
    kw = ((l >= 2).astype(jnp.int32) + (l >= 4).astype(jnp.int32)
          + (l >= 8).astype(jnp.int32))
    row_a = kw * WF + jnp.minimum(ws, WF - 1)
    row_b = kw * WF + jnp.maximum(we - (1 << kw), 0)
    pos = jnp.where(hs % 16 >= 8, hs + 56, hs)
    sh16 = pos // 16
    lo_h = pos % 16
    bins = jnp.concatenate([row_a, row_b, len_w, sh16, lo_h, lo_h + he - hs],
                           axis=1)
    bins = jnp.pad(bins, ((0, NPAD - NPROP), (0, 0))).reshape(NPAD, 1, 6 * P)

    fwhc = feat[0].transpose(2, 1, 0).astype(jnp.bfloat16)

    pooled = pl.pallas_call(
        _pool_kernel,
        grid=(NPAD // RB,),
        in_specs=[
            pl.BlockSpec((RB, 1, 6 * P), lambda i: (i, 0, 0),
                         memory_space=pltpu.SMEM),
            pl.BlockSpec((HF, WF, C), lambda i: (0, 0, 0)),
        ],
        out_specs=pl.BlockSpec((RB, P * P, C), lambda i: (i, 0, 0)),
        out_shape=jax.ShapeDtypeStruct((NPAD, P * P, C), jnp.bfloat16),
        scratch_shapes=[pltpu.VMEM((P, WP, C), jnp.bfloat16),
                        pltpu.VMEM((4 * WF, HF, C), jnp.bfloat16)],
    )(bins, fwhc)

    xc, w6q = pl.pallas_call(
        _xpose_kernel,
        grid=(NPAD // TB,),
        in_specs=[pl.BlockSpec((TB, P * P, C), lambda i: (i, 0, 0)),
                  pl.BlockSpec((TB, KDIM), lambda i: (i, 0))],
        out_specs=[pl.BlockSpec((TB, C, P * P), lambda i: (i, 0, 0)),
                   pl.BlockSpec((TB, KDIM), lambda i: (i, 0))],
        out_shape=[jax.ShapeDtypeStruct((NPAD, C, P * P), jnp.bfloat16),
                   jax.ShapeDtypeStruct((FC, KDIM), jnp.bfloat16)],
    )(pooled, W6)
    x = xc.reshape(NPAD, KDIM)

    wb_perm = Wb.reshape(NC, 4, FC).transpose(1, 0, 2).reshape(4 * NC, FC)
    wh = jnp.concatenate([Wc, wb_perm], axis=0).astype(jnp.bfloat16)
    bhv = jnp.concatenate([bc, bb.reshape(NC, 4).T.reshape(-1)])[None, :]
    prop_p = jnp.pad(proposals, ((0, NPAD - NPROP), (0, 0)))

    outs = pl.pallas_call(
        _fc_kernel,
        grid=(NPAD // BM,),
        in_specs=[
            pl.BlockSpec((BM, KDIM), lambda m: (m, 0)),
            pl.BlockSpec((FC, KDIM), lambda m: (0, 0)),
            pl.BlockSpec((1, FC), lambda m: (0, 0)),
            pl.BlockSpec((FC, FC), lambda m: (0, 0)),
            pl.BlockSpec((1, FC), lambda m: (0, 0)),
            pl.BlockSpec((5 * NC, FC), lambda m: (0, 0)),
            pl.BlockSpec((1, 5 * NC), lambda m: (0, 0)),
            pl.BlockSpec((BM, 4), lambda m: (m, 0)),
        ],
        out_specs=[pl.BlockSpec((BM, NC), lambda m: (m, 0))] * 5,
        out_shape=[jax.ShapeDtypeStruct((NPAD, NC), jnp.float32)] * 5,
    )(x, w6q, b6[None, :], W7.astype(jnp.bfloat16), b7[None, :], wh, bhv,
      prop_p)
    sc, x1o, y1o, x2o, y2o = outs

    pred_boxes = jnp.stack(
        [x1o[:NPROP], y1o[:NPROP], x2o[:NPROP], y2o[:NPROP]], axis=2)
    pred_scores = sc[:NPROP]
    return pred_boxes, pred_scores


# speedup vs baseline: 1.2637x; 1.0001x over previous
import math

import jax
import jax.numpy as jnp
from jax import lax
from jax.experimental import pallas as pl
from jax.experimental.pallas import tpu as pltpu

C = 256
HF = 50
WF = 50
P = 7
RB = 32
TB = 64
KWW = 16
WP = 128
NPROP = 1000
NPAD = 1024
NC = 91
FC = 1024
KDIM = C * P * P
BM = 256
KT = 7
BK = KDIM // KT
NEG = -1e30
BBOX_CLIP = math.log(1000.0 / 16)


def _pool_kernel(b_ref, f_ref, o_ref, out1_ref, pyr_ref):
    @pl.when(pl.program_id(0) == 0)
    def _build():
        def cp(h, _):
            pyr_ref[pl.ds(h, 1), :, :] = f_ref[pl.ds(h, 1), :, :]
            return 0
        lax.fori_loop(0, HF, cp, 0)
        for k in (1, 2, 3):
            d = 1 << (k - 1)

            def bd(h, _, base=k * HF, pbase=(k - 1) * HF, d=d):
                a = pyr_ref[pl.ds(pbase + h, 1), :, :]
                b = pyr_ref[pl.ds(pbase + jnp.minimum(h + d, HF - 1), 1), :, :]
                pyr_ref[pl.ds(base + h, 1), :, :] = jnp.maximum(a, b)
                return 0

            lax.fori_loop(0, HF, bd, 0)

    def roi_body(i, _):
        for ph in range(P):
            ra = b_ref[i, 0, ph]
            rb = b_ref[i, 0, P + ph]
            ln = b_ref[i, 0, 2 * P + ph]
            a = pyr_ref[pl.ds(ra, 1), :, :]
            b = pyr_ref[pl.ds(rb, 1), :, :]
            v = jnp.maximum(a, b)[0]
            v = jnp.where(ln > 0, v, NEG)
            out1_ref[ph, 0:HF, :] = v
            out1_ref[ph, 56:56 + HF, :] = v
        for pw in range(P):
            sw = b_ref[i, 0, 3 * P + pw] * 16
            lo = b_ref[i, 0, 4 * P + pw]
            hi = b_ref[i, 0, 5 * P + pw]
            win = out1_ref[:, pl.ds(sw, KWW), :]
            r = lax.broadcasted_iota(jnp.int32, (1, KWW, 1), 1)
            v = jnp.where((r >= lo) & (r < hi), win, NEG)
            cell = jnp.max(v, axis=1)
            cell = jnp.where(cell < -1e29, 0.0, cell)
            o_ref[pl.ds(i, 1), pw * P:(pw + 1) * P, :] = cell[None]
        return 0

    lax.fori_loop(0, RB, roi_body, 0)


def _xpose_kernel(x_ref, w_ref, o_ref, ow_ref):
    o_ref[:] = jnp.transpose(x_ref[:], (0, 2, 1))
    ow_ref[:] = w_ref[:].astype(jnp.bfloat16)


def _fc_kernel(x_ref, w6_ref, b6_ref, w7_ref, b7_ref, wh_ref, bh_ref, pr_ref,
               sc_ref, x1_ref, y1_ref, x2_ref, y2_ref):
    dn = (((1,), (1,)), ((), ()))
    h6 = lax.dot_general(x_ref[:], w6_ref[:], dn,
                         preferred_element_type=jnp.float32) + b6_ref[:]
    h6 = jnp.maximum(h6, 0.0).astype(jnp.bfloat16)
    h7 = lax.dot_general(h6, w7_ref[:], dn,
                         preferred_element_type=jnp.float32) + b7_ref[:]
    h7 = jnp.maximum(h7, 0.0).astype(jnp.bfloat16)
    head = lax.dot_general(h7, wh_ref[:], dn,
                           preferred_element_type=jnp.float32) + bh_ref[:]
    cls = head[:, 0:NC]
    mx = jnp.max(cls, axis=1, keepdims=True)
    e = jnp.exp(cls - mx)
    sc_ref[:] = e / jnp.sum(e, axis=1, keepdims=True)
    dx = head[:, NC:2 * NC]
    dy = head[:, 2 * NC:3 * NC]
    dw = jnp.minimum(head[:, 3 * NC:4 * NC], BBOX_CLIP)
    dh = jnp.minimum(head[:, 4 * NC:5 * NC], BBOX_CLIP)
    p = pr_ref[:]
    w = p[:, 2:3] - p[:, 0:1]
    h = p[:, 3:4] - p[:, 1:2]
    cx = p[:, 0:1] + w * 0.5
    cy = p[:, 1:2] + h * 0.5
    px = dx * w + cx
    py = dy * h + cy
    pw = jnp.exp(dw) * w
    ph = jnp.exp(dh) * h
    x1_ref[:] = px - 0.5 * pw
    y1_ref[:] = py - 0.5 * ph
    x2_ref[:] = px + 0.5 * pw
    y2_ref[:] = py + 0.5 * ph


def kernel(feat, proposals, image_shape, W6, b6, W7, b7, Wc, bc, Wb, bb):
    fs = jnp.array(feat.shape[-2:], dtype=jnp.float32)
    ms = image_shape.astype(jnp.float32)
    scale = (2.0 ** jnp.round(jnp.log2(fs / ms)))[0]
    r = jnp.floor(proposals * scale + 0.5)
    x1 = r[:, 0]
    y1 = r[:, 1]
    roi_w = jnp.maximum(r[:, 2] - x1 + 1.0, 1.0)
    roi_h = jnp.maximum(r[:, 3] - y1 + 1.0, 1.0)
    bw = roi_w / P
    bh = roi_h / P
    j = jnp.arange(P, dtype=jnp.float32)
    ws = jnp.clip(jnp.floor(j * bw[:, None]) + x1[:, None], 0.0, WF).astype(jnp.int32)
    we = jnp.clip(jnp.ceil((j + 1.0) * bw[:, None]) + x1[:, None], 0.0, WF).astype(jnp.int32)
    hs = jnp.clip(jnp.floor(j * bh[:, None]) + y1[:, None], 0.0, HF).astype(jnp.int32)
    he = jnp.clip(jnp.ceil((j + 1.0) * bh[:, None]) + y1[:, None], 0.0, HF).astype(jnp.int32)
    len_w = we - ws
    l = jnp.maximum(len_w, 1)
    kw = ((l >= 2).astype(jnp.int32) + (l >= 4).astype(jnp.int32)
          + (l >= 8).astype(jnp.int32))
    row_a = kw * WF + jnp.minimum(ws, WF - 1)
    row_b = kw * WF + jnp.maximum(we - (1 << kw), 0)
    pos = jnp.where(hs % 16 >= 8, hs + 56, hs)
    sh16 = pos // 16
    lo_h = pos % 16
    bins = jnp.concatenate([row_a, row_b, len_w, sh16, lo_h, lo_h + he - hs],
                           axis=1)
    bins = jnp.pad(bins, ((0, NPAD - NPROP), (0, 0))).reshape(NPAD, 1, 6 * P)

    fwhc = feat[0].transpose(2, 1, 0).astype(jnp.bfloat16)

    pooled = pl.pallas_call(
        _pool_kernel,
        grid=(NPAD // RB,),
        in_specs=[
            pl.BlockSpec((RB, 1, 6 * P), lambda i: (i, 0, 0),
                         memory_space=pltpu.SMEM),
            pl.BlockSpec((HF, WF, C), lambda i: (0, 0, 0)),
        ],
        out_specs=pl.BlockSpec((RB, P * P, C), lambda i: (i, 0, 0)),
        out_shape=jax.ShapeDtypeStruct((NPAD, P * P, C), jnp.bfloat16),
        scratch_shapes=[pltpu.VMEM((P, WP, C), jnp.bfloat16),
                        pltpu.VMEM((4 * WF, HF, C), jnp.bfloat16)],
    )(bins, fwhc)

    xc, w6q = pl.pallas_call(
        _xpose_kernel,
        grid=(NPAD // TB,),
        in_specs=[pl.BlockSpec((TB, P * P, C), lambda i: (i, 0, 0)),
                  pl.BlockSpec((TB, KDIM), lambda i: (i, 0))],
        out_specs=[pl.BlockSpec((TB, C, P * P), lambda i: (i, 0, 0)),
                   pl.BlockSpec((TB, KDIM), lambda i: (i, 0))],
        out_shape=[jax.ShapeDtypeStruct((NPAD, C, P * P), jnp.bfloat16),
                   jax.ShapeDtypeStruct((FC, KDIM), jnp.bfloat16)],
    )(pooled, W6)
    x = xc.reshape(NPAD, KDIM)

    wb_perm = Wb.reshape(NC, 4, FC).transpose(1, 0, 2).reshape(4 * NC, FC)
    wh = jnp.concatenate([Wc, wb_perm], axis=0).astype(jnp.bfloat16)
    bhv = jnp.concatenate([bc, bb.reshape(NC, 4).T.reshape(-1)])[None, :]
    prop_p = jnp.pad(proposals, ((0, NPAD - NPROP), (0, 0)))

    outs = pl.pallas_call(
        _fc_kernel,
        grid=(NPAD // BM,),
        in_specs=[
            pl.BlockSpec((BM, KDIM), lambda m: (m, 0)),
            pl.BlockSpec((FC, KDIM), lambda m: (0, 0)),
            pl.BlockSpec((1, FC), lambda m: (0, 0)),
            pl.BlockSpec((FC, FC), lambda m: (0, 0)),
            pl.BlockSpec((1, FC), lambda m: (0, 0)),
            pl.BlockSpec((5 * NC, FC), lambda m: (0, 0)),
            pl.BlockSpec((1, 5 * NC), lambda m: (0, 0)),
            pl.BlockSpec((BM, 4), lambda m: (m, 0)),
        ],
        out_specs=[pl.BlockSpec((BM, NC), lambda m: (m, 0))] * 5,
        out_shape=[jax.ShapeDtypeStruct((NPAD, NC), jnp.float32)] * 5,
    )(x, w6q, b6[None, :], W7.astype(jnp.bfloat16), b7[None, :], wh, bhv,
      prop_p)
    sc, x1o, y1o, x2o, y2o = outs

    pred_boxes = jnp.stack(
        [x1o[:NPROP], y1o[:NPROP], x2o[:NPROP], y2o[:NPROP]], axis=2)
    pred_scores = sc[:NPROP]
    return pred_boxes, pred_scores
